# baseline (speedup 1.0000x reference)
import functools

import jax
import jax.numpy as jnp
from jax import lax
from jax.experimental import pallas as pl
from jax.experimental.pallas import tpu as pltpu
from jax.experimental.pallas import tpu_sc as plsc

B = 16
T = 4096
D = 256
CHUNK = 128
NBLK = T // CHUNK
LANES = 16
NBUF = 3


@functools.lru_cache(maxsize=None)
def _build_unpack(total_rows: int):
  info = plsc.get_sparse_core_info()
  nc = info.num_cores
  mesh = plsc.VectorSubcoreMesh(core_axis_name="c", subcore_axis_name="s")

  def body(x_hbm, seq_hbm, out_hbm, seqv, idx0, idx1, idx2, buf0, buf1, buf2,
           zsh, gsem0, gsem1, gsem2, wsem0, wsem1, wsem2, zsem):
    wid = lax.axis_index("s") * nc + lax.axis_index("c")
    p = wid // 4
    q = wid % 4

    pltpu.sync_copy(seq_hbm, seqv)
    iota = lax.iota(jnp.int32, LANES)
    lenv = seqv[...]
    lens = [lenv[bb] for bb in range(B)]

    def sel_len(bidx):
      return functools.reduce(
          lambda acc, bb: jnp.where(bidx == bb, lens[bb], acc), range(B),
          jnp.int32(0))

    b1 = p
    b2 = B - 1 - p
    len1 = sel_len(b1)
    len2 = sel_len(b2)

    nb1 = len1 // CHUNK
    rem1 = len1 - nb1 * CHUNK
    nb2 = len2 // CHUNK
    rem2 = len2 - nb2 * CHUNK
    nv1 = jnp.maximum(0, (nb1 - q + 3) // 4)
    nv2 = jnp.maximum(0, (nb2 - q + 3) // 4)
    nv = nv1 + nv2

    idxs = (idx0, idx1, idx2)
    bufs = (buf0, buf1, buf2)
    gsems = (gsem0, gsem1, gsem2)
    wsems = (wsem0, wsem1, wsem2)
    zero16 = jnp.zeros((LANES,), jnp.float32)

    def build_idx(t_base, badd, idxref):
      for j in range(CHUNK // LANES):
        tvec = t_base + j * LANES + iota
        off = jnp.zeros((LANES,), jnp.int32)
        for bb in range(B):
          off = off + jnp.minimum(lens[bb], tvec)
        idxref[pl.ds(j * LANES, LANES)] = jnp.minimum(off + badd,
                                                      total_rows - 1)

    def task(i):
      in1 = i < nv1
      bsel = jnp.where(in1, b1, b2)
      j = jnp.where(in1, i, i - nv1)
      t_base = (q + 4 * j) * CHUNK
      return bsel, t_base

    def launch(i, k):
      bsel, t_base = task(i)
      build_idx(t_base, bsel, idxs[k])
      pltpu.async_copy(x_hbm.at[idxs[k]], bufs[k], gsems[k])

    @pl.when(nv > 0)
    def _prime():
      launch(0, 0)

    @pl.when(lax.axis_index("s") == 0)
    def _seed_zeros():
      def zrow(i, carry):
        for k in range(D // LANES):
          buf2[i, pl.ds(k * LANES, LANES)] = zero16
        return carry

      lax.fori_loop(0, CHUNK, zrow, 0)
      pltpu.sync_copy(buf2, zsh)

    plsc.subcore_barrier()

    zb1 = nb1 + jnp.where(rem1 > 0, 1, 0)
    zb2 = nb2 + jnp.where(rem2 > 0, 1, 0)
    jz1 = jnp.clip((zb1 - q + 3) // 4, 0, NBLK // 4)
    jz2 = jnp.clip((zb2 - q + 3) // 4, 0, NBLK // 4)
    nzq1 = NBLK // 4 - jz1
    nzq2 = NBLK // 4 - jz2

    def make_zfire(row_base, jz):
      def zfire(j, carry):
        col = q + 4 * (jz + j)
        pltpu.async_copy(
            zsh, out_hbm.at[pl.ds(row_base + col * CHUNK, CHUNK)], zsem)
        return carry
      return zfire

    lax.fori_loop(0, nzq1, make_zfire(b1 * T, jz1), 0)
    lax.fori_loop(0, nzq2, make_zfire(b2 * T, jz2), 0)

    def wwait(k):
      pltpu.make_async_copy(bufs[k], out_hbm.at[pl.ds(0, CHUNK)],
                            wsems[k]).wait()

    def gring(i3, carry):
      for k in range(NBUF):
        i = i3 * NBUF + k

        @pl.when(i < nv)
        def _step():
          nxt = i + 1
          kn = (k + 1) % NBUF

          @pl.when(nxt < nv)
          def _build_next():
            bsel, t_base = task(nxt)
            build_idx(t_base, bsel, idxs[kn])

          pltpu.make_async_copy(x_hbm.at[idxs[k]], bufs[k], gsems[k]).wait()
          bsel, t_base = task(i)
          pltpu.async_copy(bufs[k],
                           out_hbm.at[pl.ds(bsel * T + t_base, CHUNK)],
                           wsems[k])

          @pl.when(nxt < nv)
          def _launch_next():
            @pl.when(i >= NBUF - 1)
            def _reuse_wait():
              wwait(kn)

            pltpu.async_copy(x_hbm.at[idxs[kn]], bufs[kn], gsems[kn])

      return carry

    lax.fori_loop(0, (nv + NBUF - 1) // NBUF, gring, 0)

    for k in range(NBUF):
      pending = functools.reduce(
          lambda acc, m: acc | ((nv >= m) & ((nv - m) % NBUF == k)),
          range(1, NBUF + 1), jnp.bool_(False))

      @pl.when(pending)
      def _drain():
        wwait(k)

    def partial(bsel, len_b, nb, rem):
      @pl.when((rem > 0) & (q == nb % 4))
      def _partial():
        t_base = nb * CHUNK
        build_idx(t_base, bsel, idx0)
        pltpu.async_copy(x_hbm.at[idx0], buf0, gsem0).wait()

        def ztail(r, carry):
          for k in range(D // LANES):
            buf0[r, pl.ds(k * LANES, LANES)] = zero16
          return carry

        lax.fori_loop(rem, CHUNK, ztail, 0)
        pltpu.sync_copy(buf0, out_hbm.at[pl.ds(bsel * T + t_base, CHUNK)])

    partial(b1, len1, nb1, rem1)
    partial(b2, len2, nb2, rem2)

    def zdrain(i, carry):
      pltpu.make_async_copy(zsh, out_hbm.at[pl.ds(0, CHUNK)], zsem).wait()
      return carry

    lax.fori_loop(0, nzq1 + nzq2, zdrain, 0)

  return pl.kernel(
      body,
      out_type=jax.ShapeDtypeStruct((B * T, D), jnp.float32),
      mesh=mesh,
      scratch_types=[
          pltpu.VMEM((LANES,), jnp.int32),
          pltpu.VMEM((CHUNK,), jnp.int32),
          pltpu.VMEM((CHUNK,), jnp.int32),
          pltpu.VMEM((CHUNK,), jnp.int32),
          pltpu.VMEM((CHUNK, D), jnp.float32),
          pltpu.VMEM((CHUNK, D), jnp.float32),
          pltpu.VMEM((CHUNK, D), jnp.float32),
          pltpu.VMEM_SHARED((CHUNK, D), jnp.float32),
          pltpu.SemaphoreType.DMA,
          pltpu.SemaphoreType.DMA,
          pltpu.SemaphoreType.DMA,
          pltpu.SemaphoreType.DMA,
          pltpu.SemaphoreType.DMA,
          pltpu.SemaphoreType.DMA,
          pltpu.SemaphoreType.DMA,
      ],
  )


def kernel(x, seq_len, mask_times, mask_features):
  out_flat = _build_unpack(x.shape[0])(x, seq_len.astype(jnp.int32))
  return (out_flat.reshape(B, T, D), seq_len, mask_times, mask_features)


# speedup vs baseline: 1.0190x; 1.0190x over previous
import functools

import jax
import jax.numpy as jnp
from jax import lax
from jax.experimental import pallas as pl
from jax.experimental.pallas import tpu as pltpu
from jax.experimental.pallas import tpu_sc as plsc

B = 16
T = 4096
D = 256
CHUNK = 128
NBLK = T // CHUNK
LANES = 16
NBUF = 3
ZROWS = 64


@functools.lru_cache(maxsize=None)
def _build_unpack(total_rows: int):
  info = plsc.get_sparse_core_info()
  nc = info.num_cores
  mesh = plsc.VectorSubcoreMesh(core_axis_name="c", subcore_axis_name="s")

  def body(x_hbm, seq_hbm, out_hbm, seqv, idx0, idx1, idx2, buf0, buf1, buf2,
           zbuf, gsem0, gsem1, gsem2, wsem0, wsem1, wsem2, zsem):
    wid = lax.axis_index("s") * nc + lax.axis_index("c")
    p = wid // 4
    q = wid % 4

    pltpu.sync_copy(seq_hbm, seqv)
    iota = lax.iota(jnp.int32, LANES)
    lenv = seqv[...]
    lens = [lenv[bb] for bb in range(B)]

    def sel_len(bidx):
      return functools.reduce(
          lambda acc, bb: jnp.where(bidx == bb, lens[bb], acc), range(B),
          jnp.int32(0))

    b1 = p
    b2 = B - 1 - p
    len1 = sel_len(b1)
    len2 = sel_len(b2)

    nb1 = len1 // CHUNK
    rem1 = len1 - nb1 * CHUNK
    nb2 = len2 // CHUNK
    rem2 = len2 - nb2 * CHUNK
    nv1 = jnp.maximum(0, (nb1 - q + 3) // 4)
    nv2 = jnp.maximum(0, (nb2 - q + 3) // 4)
    nv = nv1 + nv2

    idxs = (idx0, idx1, idx2)
    bufs = (buf0, buf1, buf2)
    gsems = (gsem0, gsem1, gsem2)
    wsems = (wsem0, wsem1, wsem2)
    zero16 = jnp.zeros((LANES,), jnp.float32)

    def build_idx(t_base, badd, idxref):
      for j in range(CHUNK // LANES):
        tvec = t_base + j * LANES + iota
        off = jnp.zeros((LANES,), jnp.int32)
        for bb in range(B):
          off = off + jnp.minimum(lens[bb], tvec)
        idxref[pl.ds(j * LANES, LANES)] = jnp.minimum(off + badd,
                                                      total_rows - 1)

    def task(i):
      in1 = i < nv1
      bsel = jnp.where(in1, b1, b2)
      j = jnp.where(in1, i, i - nv1)
      t_base = (q + 4 * j) * CHUNK
      return bsel, t_base

    def launch(i, k):
      bsel, t_base = task(i)
      build_idx(t_base, bsel, idxs[k])
      pltpu.async_copy(x_hbm.at[idxs[k]], bufs[k], gsems[k])

    @pl.when(nv > 0)
    def _prime():
      launch(0, 0)

    def zrow(i, carry):
      for k in range(D // LANES):
        zbuf[i, pl.ds(k * LANES, LANES)] = zero16
      return carry

    lax.fori_loop(0, ZROWS, zrow, 0)

    zb1 = nb1 + jnp.where(rem1 > 0, 1, 0)
    zb2 = nb2 + jnp.where(rem2 > 0, 1, 0)
    jz1 = jnp.clip((zb1 - q + 3) // 4, 0, NBLK // 4)
    jz2 = jnp.clip((zb2 - q + 3) // 4, 0, NBLK // 4)
    nzq1 = NBLK // 4 - jz1
    nzq2 = NBLK // 4 - jz2

    def make_zfire(row_base, jz):
      def zfire(j, carry):
        col = q + 4 * (jz + j // 2)
        pltpu.async_copy(
            zbuf,
            out_hbm.at[pl.ds(row_base + col * CHUNK + (j % 2) * ZROWS,
                             ZROWS)],
            zsem)
        return carry
      return zfire

    lax.fori_loop(0, 2 * nzq1, make_zfire(b1 * T, jz1), 0)
    lax.fori_loop(0, 2 * nzq2, make_zfire(b2 * T, jz2), 0)

    def wwait(k):
      pltpu.make_async_copy(bufs[k], out_hbm.at[pl.ds(0, CHUNK)],
                            wsems[k]).wait()

    def gring(i3, carry):
      for k in range(NBUF):
        i = i3 * NBUF + k

        @pl.when(i < nv)
        def _step():
          nxt = i + 1
          kn = (k + 1) % NBUF

          @pl.when(nxt < nv)
          def _launch_next():
            bsel, t_base = task(nxt)
            build_idx(t_base, bsel, idxs[kn])

            @pl.when(i >= NBUF - 1)
            def _reuse_wait():
              wwait(kn)

            pltpu.async_copy(x_hbm.at[idxs[kn]], bufs[kn], gsems[kn])

          pltpu.make_async_copy(x_hbm.at[idxs[k]], bufs[k], gsems[k]).wait()
          bsel, t_base = task(i)
          pltpu.async_copy(bufs[k],
                           out_hbm.at[pl.ds(bsel * T + t_base, CHUNK)],
                           wsems[k])

      return carry

    lax.fori_loop(0, (nv + NBUF - 1) // NBUF, gring, 0)

    for k in range(NBUF):
      pending = functools.reduce(
          lambda acc, m: acc | ((nv >= m) & ((nv - m) % NBUF == k)),
          range(1, NBUF + 1), jnp.bool_(False))

      @pl.when(pending)
      def _drain():
        wwait(k)

    def partial(bsel, len_b, nb, rem):
      @pl.when((rem > 0) & (q == nb % 4))
      def _partial():
        t_base = nb * CHUNK
        build_idx(t_base, bsel, idx0)
        pltpu.async_copy(x_hbm.at[idx0], buf0, gsem0).wait()

        def ztail(r, carry):
          for k in range(D // LANES):
            buf0[r, pl.ds(k * LANES, LANES)] = zero16
          return carry

        lax.fori_loop(rem, CHUNK, ztail, 0)
        pltpu.sync_copy(buf0, out_hbm.at[pl.ds(bsel * T + t_base, CHUNK)])

    partial(b1, len1, nb1, rem1)
    partial(b2, len2, nb2, rem2)

    def zdrain(i, carry):
      pltpu.make_async_copy(zbuf, out_hbm.at[pl.ds(0, ZROWS)], zsem).wait()
      return carry

    lax.fori_loop(0, 2 * (nzq1 + nzq2), zdrain, 0)

  return pl.kernel(
      body,
      out_type=jax.ShapeDtypeStruct((B * T, D), jnp.float32),
      mesh=mesh,
      scratch_types=[
          pltpu.VMEM((LANES,), jnp.int32),
          pltpu.VMEM((CHUNK,), jnp.int32),
          pltpu.VMEM((CHUNK,), jnp.int32),
          pltpu.VMEM((CHUNK,), jnp.int32),
          pltpu.VMEM((CHUNK, D), jnp.float32),
          pltpu.VMEM((CHUNK, D), jnp.float32),
          pltpu.VMEM((CHUNK, D), jnp.float32),
          pltpu.VMEM((ZROWS, D), jnp.float32),
          pltpu.SemaphoreType.DMA,
          pltpu.SemaphoreType.DMA,
          pltpu.SemaphoreType.DMA,
          pltpu.SemaphoreType.DMA,
          pltpu.SemaphoreType.DMA,
          pltpu.SemaphoreType.DMA,
          pltpu.SemaphoreType.DMA,
      ],
  )


def kernel(x, seq_len, mask_times, mask_features):
  out_flat = _build_unpack(x.shape[0])(x, seq_len.astype(jnp.int32))
  return (out_flat.reshape(B, T, D), seq_len, mask_times, mask_features)
